# baseline (speedup 1.0000x reference)
import functools

import jax
import jax.numpy as jnp
from jax import lax
from jax.experimental import pallas as pl
from jax.experimental.pallas import tpu as pltpu
from jax.experimental.pallas import tpu_sc as plsc

N = 16384
D = 64
K = 1024
BLK = 1024
COMMIT = 0.25
IDX_CHUNK = 128


def _tc_body(x_ref, xs_ref, m2e_ref, esq_ref, idx_ref, stats_ref, loss_acc):
    i = pl.program_id(0)

    @pl.when(i == 0)
    def _init():
        loss_acc[0] = 0.0

    x = x_ref[...]
    m2e = m2e_ref[...]
    scores_t = lax.dot_general(
        m2e, x, dimension_numbers=(((1,), (1,)), ((), ())),
        preferred_element_type=jnp.float32)
    x_sq = xs_ref[...][None, :]
    e_sq = esq_ref[...]
    dist = (x_sq + e_sq) + scores_t
    m = jnp.min(dist, axis=0, keepdims=True)
    iota = lax.broadcasted_iota(jnp.int32, dist.shape, 0)
    idx = jnp.min(jnp.where(dist == m, iota, K), axis=0)
    idx = jnp.minimum(idx, K - 1)
    idx_ref[...] = idx

    loss_acc[0] += jnp.sum(m)

    @pl.when(i == pl.num_programs(0) - 1)
    def _fin():
        msq = loss_acc[0] / (N * D)
        stats_ref[0] = (1.0 + COMMIT) * msq
        stats_ref[1] = msq
        stats_ref[2] = msq


def _perp_body(idx_ref, perp_ref, counts_acc):
    i = pl.program_id(0)

    @pl.when(i == 0)
    def _init():
        counts_acc[...] = jnp.zeros_like(counts_acc)

    idx = idx_ref[...]
    iota = lax.broadcasted_iota(jnp.int32, (K, BLK), 0)
    onehot = (iota == idx[None, :]).astype(jnp.float32)
    counts_acc[...] += jnp.sum(onehot, axis=1, keepdims=True)

    @pl.when(i == pl.num_programs(0) - 1)
    def _fin():
        avg = counts_acc[...] / N
        ent = jnp.sum(avg * jnp.log(avg + 1e-10), axis=0, keepdims=True)
        perp = jnp.exp(-ent)
        perp_ref[0] = perp[0, 0]


def _perp_call(idx):
    grid = N // BLK
    return pl.pallas_call(
        _perp_body,
        grid=(grid,),
        in_specs=[pl.BlockSpec((BLK,), lambda i: (i,))],
        out_specs=pl.BlockSpec(memory_space=pltpu.SMEM),
        out_shape=jax.ShapeDtypeStruct((1,), jnp.float32),
        scratch_shapes=[pltpu.VMEM((K, 1), jnp.float32)],
        compiler_params=pltpu.CompilerParams(
            dimension_semantics=("arbitrary",)),
    )(idx)


def _tc_call(inputs, xs, m2e, esq):
    grid = N // BLK
    return pl.pallas_call(
        _tc_body,
        grid=(grid,),
        in_specs=[
            pl.BlockSpec((BLK, D), lambda i: (i, 0)),
            pl.BlockSpec((BLK,), lambda i: (i,)),
            pl.BlockSpec((K, D), lambda i: (0, 0)),
            pl.BlockSpec((K, 1), lambda i: (0, 0)),
        ],
        out_specs=[
            pl.BlockSpec((BLK,), lambda i: (i,)),
            pl.BlockSpec(memory_space=pltpu.SMEM),
        ],
        out_shape=[
            jax.ShapeDtypeStruct((N,), jnp.int32),
            jax.ShapeDtypeStruct((3,), jnp.float32),
        ],
        scratch_shapes=[
            pltpu.SMEM((1,), jnp.float32),
        ],
        compiler_params=pltpu.CompilerParams(
            dimension_semantics=("arbitrary",)),
    )(inputs, xs, m2e, esq)


@functools.cache
def _sc_gather_call():
    info = plsc.get_sparse_core_info()
    nw = info.num_cores * info.num_subcores
    b_per_w = N // nw
    chunks = b_per_w // IDX_CHUNK
    nc = info.num_cores
    mesh = plsc.VectorSubcoreMesh(core_axis_name="c", subcore_axis_name="s")

    @functools.partial(
        pl.kernel,
        mesh=mesh,
        out_type=jax.ShapeDtypeStruct((N, D), jnp.float32),
        scratch_types=[
            pltpu.VMEM((chunks, IDX_CHUNK), jnp.int32),
            pltpu.VMEM((b_per_w, D), jnp.float32),
            pltpu.SemaphoreType.DMA,
        ],
        compiler_params=pltpu.CompilerParams(use_tc_tiling_on_sc=False),
    )
    def gather(emb_hbm, idx_hbm, out_hbm, idx_v, rows_v, sem):
        wid = lax.axis_index("s") * nc + lax.axis_index("c")
        base = wid * b_per_w
        pltpu.sync_copy(idx_hbm.at[wid], idx_v)
        copies = [
            pltpu.async_copy(
                emb_hbm.at[idx_v.at[j]],
                rows_v.at[pl.ds(j * IDX_CHUNK, IDX_CHUNK)],
                sem,
            )
            for j in range(chunks)
        ]
        for cp in copies:
            cp.wait()
        pltpu.sync_copy(rows_v, out_hbm.at[pl.ds(base, b_per_w)])

    return gather, nw, chunks


def kernel(inputs, embeddings):
    xs = jnp.sum(inputs ** 2, axis=1)
    esq = jnp.sum(embeddings ** 2, axis=1).reshape(K, 1)
    m2e = embeddings * (-2.0)
    idx, stats = _tc_call(inputs, xs, m2e, esq)
    gather, nw, chunks = _sc_gather_call()
    quantized = gather(embeddings, idx.reshape(nw, chunks, IDX_CHUNK))
    perp = _perp_call(idx)
    return (quantized, idx, stats[0], stats[1], stats[2], perp[0])


# speedup vs baseline: 1.0762x; 1.0762x over previous
import functools

import jax
import jax.numpy as jnp
from jax import lax
from jax.experimental import pallas as pl
from jax.experimental.pallas import tpu as pltpu
from jax.experimental.pallas import tpu_sc as plsc

N = 16384
D = 64
K = 1024
BLK = 2048
COMMIT = 0.25
IDX_CHUNK = 128
PBLK = 4096


def _tc_body(x_ref, xs_ref, m2e_ref, esq_ref, idx_ref, stats_ref, loss_acc):
    i = pl.program_id(0)

    @pl.when(i == 0)
    def _init():
        loss_acc[0] = 0.0

    x = x_ref[...]
    m2e = m2e_ref[...]
    scores_t = lax.dot_general(
        m2e, x, dimension_numbers=(((1,), (1,)), ((), ())),
        preferred_element_type=jnp.float32)
    x_sq = xs_ref[...][None, :]
    e_sq = esq_ref[...]
    dist = (x_sq + e_sq) + scores_t
    m = jnp.min(dist, axis=0, keepdims=True)
    iota = lax.broadcasted_iota(jnp.int32, dist.shape, 0)
    idx = jnp.min(jnp.where(dist == m, iota, K), axis=0)
    idx = jnp.minimum(idx, K - 1)
    idx_ref[...] = idx

    loss_acc[0] += jnp.sum(m)

    @pl.when(i == pl.num_programs(0) - 1)
    def _fin():
        msq = loss_acc[0] / (N * D)
        stats_ref[0] = (1.0 + COMMIT) * msq
        stats_ref[1] = msq
        stats_ref[2] = msq


def _perp_body(idx_ref, perp_ref, counts_acc):
    i = pl.program_id(0)

    @pl.when(i == 0)
    def _init():
        counts_acc[...] = jnp.zeros_like(counts_acc)

    idx = idx_ref[...]
    iota = lax.broadcasted_iota(jnp.int32, (K, PBLK), 0)
    onehot = (iota == idx[None, :]).astype(jnp.float32)
    counts_acc[...] += jnp.sum(onehot, axis=1, keepdims=True)

    @pl.when(i == pl.num_programs(0) - 1)
    def _fin():
        avg = counts_acc[...] / N
        ent = jnp.sum(avg * jnp.log(avg + 1e-10), axis=0, keepdims=True)
        perp = jnp.exp(-ent)
        perp_ref[0] = perp[0, 0]


def _perp_call(idx):
    grid = N // PBLK
    return pl.pallas_call(
        _perp_body,
        grid=(grid,),
        in_specs=[pl.BlockSpec((PBLK,), lambda i: (i,))],
        out_specs=pl.BlockSpec(memory_space=pltpu.SMEM),
        out_shape=jax.ShapeDtypeStruct((1,), jnp.float32),
        scratch_shapes=[pltpu.VMEM((K, 1), jnp.float32)],
        compiler_params=pltpu.CompilerParams(
            dimension_semantics=("arbitrary",)),
    )(idx)


def _tc_call(inputs, xs, m2e, esq):
    grid = N // BLK
    return pl.pallas_call(
        _tc_body,
        grid=(grid,),
        in_specs=[
            pl.BlockSpec((BLK, D), lambda i: (i, 0)),
            pl.BlockSpec((BLK,), lambda i: (i,)),
            pl.BlockSpec((K, D), lambda i: (0, 0)),
            pl.BlockSpec((K, 1), lambda i: (0, 0)),
        ],
        out_specs=[
            pl.BlockSpec((BLK,), lambda i: (i,)),
            pl.BlockSpec(memory_space=pltpu.SMEM),
        ],
        out_shape=[
            jax.ShapeDtypeStruct((N,), jnp.int32),
            jax.ShapeDtypeStruct((3,), jnp.float32),
        ],
        scratch_shapes=[
            pltpu.SMEM((1,), jnp.float32),
        ],
        compiler_params=pltpu.CompilerParams(
            dimension_semantics=("arbitrary",)),
    )(inputs, xs, m2e, esq)


@functools.cache
def _sc_gather_call():
    info = plsc.get_sparse_core_info()
    nw = info.num_cores * info.num_subcores
    b_per_w = N // nw
    chunks = b_per_w // IDX_CHUNK
    nc = info.num_cores
    mesh = plsc.VectorSubcoreMesh(core_axis_name="c", subcore_axis_name="s")

    @functools.partial(
        pl.kernel,
        mesh=mesh,
        out_type=jax.ShapeDtypeStruct((N, D), jnp.float32),
        scratch_types=[
            pltpu.VMEM((chunks, IDX_CHUNK), jnp.int32),
            pltpu.VMEM((b_per_w, D), jnp.float32),
            pltpu.SemaphoreType.DMA,
        ],
        compiler_params=pltpu.CompilerParams(use_tc_tiling_on_sc=False),
    )
    def gather(emb_hbm, idx_hbm, out_hbm, idx_v, rows_v, sem):
        wid = lax.axis_index("s") * nc + lax.axis_index("c")
        base = wid * b_per_w
        pltpu.sync_copy(idx_hbm.at[wid], idx_v)
        copies = [
            pltpu.async_copy(
                emb_hbm.at[idx_v.at[j]],
                rows_v.at[pl.ds(j * IDX_CHUNK, IDX_CHUNK)],
                sem,
            )
            for j in range(chunks)
        ]
        for cp in copies:
            cp.wait()
        pltpu.sync_copy(rows_v, out_hbm.at[pl.ds(base, b_per_w)])

    return gather, nw, chunks


def kernel(inputs, embeddings):
    xs = jnp.sum(inputs ** 2, axis=1)
    esq = jnp.sum(embeddings ** 2, axis=1).reshape(K, 1)
    m2e = embeddings * (-2.0)
    idx, stats = _tc_call(inputs, xs, m2e, esq)
    gather, nw, chunks = _sc_gather_call()
    quantized = gather(embeddings, idx.reshape(nw, chunks, IDX_CHUNK))
    perp = _perp_call(idx)
    return (quantized, idx, stats[0], stats[1], stats[2], perp[0])
